# baseline (speedup 1.0000x reference)
import functools

import jax
import jax.numpy as jnp
from jax import lax
from jax.experimental import pallas as pl
from jax.experimental.pallas import tpu as pltpu
from jax.experimental.pallas import tpu_sc as plsc

_H = 32
_C = 3
_S = 67


def _i32(x):
    return jnp.full((16,), x, jnp.int32)


@functools.cache
def _bias_fn():
    mesh = plsc.VectorSubcoreMesh(core_axis_name="c", subcore_axis_name="s")

    @functools.partial(
        pl.kernel,
        out_type=jax.ShapeDtypeStruct((_H, _S, _S), jnp.float32),
        mesh=mesh,
        scratch_types=[
            pltpu.VMEM((15, 15), jnp.float32),
            pltpu.VMEM((_C, 64), jnp.float32),
            pltpu.VMEM((64, _C), jnp.float32),
            pltpu.VMEM((_C, _C), jnp.float32),
            pltpu.VMEM((_S, _S), jnp.float32),
            pltpu.SemaphoreType.DMA,
            pltpu.SemaphoreType.DMA,
            pltpu.SemaphoreType.DMA,
            pltpu.SemaphoreType.DMA,
        ],
        compiler_params=pltpu.CompilerParams(
            needs_layout_passes=False, use_tc_tiling_on_sc=False),
    )
    def _bias(rel_hbm, csb_hbm, scb_hbm, ccb_hbm, out_hbm,
              rel_v, csb_v, scb_v, ccb_v, out_v, sem0, sem1, sem2, sem3):
        num_cores = lax.axis_size("c")
        h = lax.axis_index("s") * num_cores + lax.axis_index("c")
        cp_rel = pltpu.make_async_copy(rel_hbm.at[h], rel_v, sem0)
        cp_csb = pltpu.make_async_copy(csb_hbm.at[h], csb_v, sem1)
        cp_scb = pltpu.make_async_copy(scb_hbm.at[h], scb_v, sem2)
        cp_ccb = pltpu.make_async_copy(ccb_hbm.at[h], ccb_v, sem3)
        cp_rel.start()
        cp_csb.start()
        cp_scb.start()
        cp_ccb.start()

        lane = lax.iota(jnp.int32, 16)

        cp_rel.wait()
        for i in range(64):
            for v in range(4):
                j = lane + (16 * v)
                dr = _i32((i >> 3) + 7) - (j >> 3)
                df = _i32((i & 7) + 7) - (j & 7)
                vals = plsc.load_gather(rel_v, [dr, df])
                plsc.store_scatter(
                    out_v, [_i32(_C + i), j + _C], vals)

        cp_csb.wait()
        for r in range(_C):
            for v in range(4):
                vals = csb_v[r, pl.ds(16 * v, 16)]
                plsc.store_scatter(
                    out_v, [_i32(r), lane + (_C + 16 * v)], vals)

        cp_scb.wait()
        for v in range(12):
            s = lane + (16 * v)
            q = (s * 21846) >> 16
            c = s - q * 3
            vals = plsc.load_gather(scb_v, [q, c])
            plsc.store_scatter(out_v, [q + _C, c], vals)

        cp_ccb.wait()
        s = lane
        q = (s * 21846) >> 16
        c = s - q * 3
        m = s < 9
        vals = plsc.load_gather(ccb_v, [q, c], mask=m)
        plsc.store_scatter(out_v, [q, c], vals, mask=m)

        pltpu.sync_copy(out_v, out_hbm.at[h])

    return _bias


def kernel(rel_bias, context_sq_bias, sq_context_bias, context_context_bias):
    return _bias_fn()(
        rel_bias, context_sq_bias, sq_context_bias, context_context_bias)
